# Initial kernel scaffold; baseline (speedup 1.0000x reference)
import jax
import jax.numpy as jnp
from jax.experimental import pallas as pl


def kernel(species, constant):
    raise NotImplementedError("write your pallas kernel here")


# speedup vs baseline: 441.0682x; 441.0682x over previous
import functools

import jax
import jax.numpy as jnp
from jax import lax
from jax.experimental import pallas as pl
from jax.experimental.pallas import tpu as pltpu
from jax.experimental.pallas import tpu_sc as plsc

N_ATOMS = 4194304
TABLE_PAD = 128
L = 16
NC = 2
NS = 16
NW = NC * NS
PER_W = N_ATOMS // NW
CHUNK = 16384
NCHUNK = PER_W // CHUNK

_mesh = plsc.VectorSubcoreMesh(core_axis_name="c", subcore_axis_name="s")


@functools.partial(
    pl.kernel,
    mesh=_mesh,
    out_type=jax.ShapeDtypeStruct((N_ATOMS,), jnp.float32),
    scratch_types=[
        pltpu.VMEM((TABLE_PAD,), jnp.float32),
        pltpu.VMEM((CHUNK,), jnp.int32),
        pltpu.VMEM((CHUNK,), jnp.int32),
        pltpu.VMEM((CHUNK,), jnp.float32),
        pltpu.VMEM((CHUNK,), jnp.float32),
        pltpu.SemaphoreType.DMA,
        pltpu.SemaphoreType.DMA,
        pltpu.SemaphoreType.DMA,
        pltpu.SemaphoreType.DMA,
    ],
    compiler_params=pltpu.CompilerParams(needs_layout_passes=False),
)
def _lookup(species_hbm, const_hbm, out_hbm,
            table_v, idx0, idx1, val0, val1,
            sin0, sin1, sout0, sout1):
    wid = lax.axis_index("s") * NC + lax.axis_index("c")
    base = wid * PER_W

    pltpu.sync_copy(const_hbm, table_v)

    idx = (idx0, idx1)
    val = (val0, val1)
    sin = (sin0, sin1)
    sout = (sout0, sout1)

    def in_copy(c, b):
        return pltpu.make_async_copy(
            species_hbm.at[pl.ds(base + c * CHUNK, CHUNK)], idx[b], sin[b])

    def out_copy(c, b):
        return pltpu.make_async_copy(
            val[b], out_hbm.at[pl.ds(base + c * CHUNK, CHUNK)], sout[b])

    def compute(b):
        idx_ref = idx[b]
        val_ref = val[b]

        def body(i, carry):
            sl = pl.ds(i * L, L)
            val_ref[sl] = plsc.load_gather(table_v, [idx_ref[sl]])
            return carry

        lax.fori_loop(0, CHUNK // L, body, 0, unroll=8)

    in_copy(0, 0).start()
    in_copy(1, 1).start()
    for c in range(NCHUNK):
        b = c & 1
        in_copy(c, b).wait()
        if c >= 2:
            out_copy(c - 2, b).wait()
        compute(b)
        out_copy(c, b).start()
        if c + 2 < NCHUNK:
            in_copy(c + 2, b).start()
    out_copy(NCHUNK - 2, 0).wait()
    out_copy(NCHUNK - 1, 1).wait()


def kernel(species, constant):
    const_padded = jnp.pad(constant, (0, TABLE_PAD - constant.shape[0]))
    return _lookup(species, const_padded)


# speedup vs baseline: 1150.4211x; 2.6083x over previous
import functools

import jax
import jax.numpy as jnp
from jax import lax
from jax.experimental import pallas as pl
from jax.experimental.pallas import tpu as pltpu
from jax.experimental.pallas import tpu_sc as plsc

N_ATOMS = 4194304
TABLE_PAD = 128
L = 16
NC = 2
NS = 16
NW = NC * NS
PER_W = N_ATOMS // NW
CHUNK = 16384
NCHUNK = PER_W // CHUNK

_mesh = plsc.VectorSubcoreMesh(core_axis_name="c", subcore_axis_name="s")


@functools.partial(
    pl.kernel,
    mesh=_mesh,
    out_type=jax.ShapeDtypeStruct((N_ATOMS,), jnp.float32),
    scratch_types=[
        pltpu.VMEM((TABLE_PAD,), jnp.float32),
        pltpu.VMEM((CHUNK,), jnp.int32),
        pltpu.VMEM((CHUNK,), jnp.int32),
        pltpu.VMEM((CHUNK,), jnp.float32),
        pltpu.VMEM((CHUNK,), jnp.float32),
        pltpu.SemaphoreType.DMA,
        pltpu.SemaphoreType.DMA,
        pltpu.SemaphoreType.DMA,
        pltpu.SemaphoreType.DMA,
    ],
    compiler_params=pltpu.CompilerParams(needs_layout_passes=False),
)
def _lookup(species_hbm, const_hbm, out_hbm,
            table_v, idx0, idx1, val0, val1,
            sin0, sin1, sout0, sout1):
    wid = lax.axis_index("s") * NC + lax.axis_index("c")
    base = wid * PER_W

    pltpu.sync_copy(const_hbm, table_v)

    idx = (idx0, idx1)
    val = (val0, val1)
    sin = (sin0, sin1)
    sout = (sout0, sout1)

    def in_copy(c, b):
        return pltpu.make_async_copy(
            species_hbm.at[pl.ds(base + c * CHUNK, CHUNK)], idx[b], sin[b])

    def out_copy(c, b):
        return pltpu.make_async_copy(
            val[b], out_hbm.at[pl.ds(base + c * CHUNK, CHUNK)], sout[b])

    def compute(b):
        idx_ref = idx[b]
        val_ref = val[b]

        @plsc.parallel_loop(0, CHUNK, L, unroll=8)
        def _(i):
            sl = pl.ds(i, L)
            val_ref[sl] = plsc.load_gather(table_v, [idx_ref[sl]])

    in_copy(0, 0).start()
    in_copy(1, 1).start()
    for c in range(NCHUNK):
        b = c & 1
        in_copy(c, b).wait()
        if c >= 2:
            out_copy(c - 2, b).wait()
        compute(b)
        out_copy(c, b).start()
        if c + 2 < NCHUNK:
            in_copy(c + 2, b).start()
    out_copy(NCHUNK - 2, 0).wait()
    out_copy(NCHUNK - 1, 1).wait()


def kernel(species, constant):
    const_padded = jnp.pad(constant, (0, TABLE_PAD - constant.shape[0]))
    return _lookup(species, const_padded)


# speedup vs baseline: 1155.7972x; 1.0047x over previous
import functools

import jax
import jax.numpy as jnp
from jax import lax
from jax.experimental import pallas as pl
from jax.experimental.pallas import tpu as pltpu
from jax.experimental.pallas import tpu_sc as plsc

N_ATOMS = 4194304
TABLE_LEN = 119
L = 16
NC = 2
NS = 16
NW = NC * NS
PER_W = N_ATOMS // NW
CHUNK = 16384
NCHUNK = PER_W // CHUNK

_mesh = plsc.VectorSubcoreMesh(core_axis_name="c", subcore_axis_name="s")


@functools.partial(
    pl.kernel,
    mesh=_mesh,
    out_type=jax.ShapeDtypeStruct((N_ATOMS,), jnp.float32),
    scratch_types=[
        pltpu.VMEM((TABLE_LEN,), jnp.float32),
        pltpu.VMEM((CHUNK,), jnp.int32),
        pltpu.VMEM((CHUNK,), jnp.int32),
        pltpu.VMEM((CHUNK,), jnp.float32),
        pltpu.VMEM((CHUNK,), jnp.float32),
        pltpu.SemaphoreType.DMA,
        pltpu.SemaphoreType.DMA,
        pltpu.SemaphoreType.DMA,
        pltpu.SemaphoreType.DMA,
    ],
    compiler_params=pltpu.CompilerParams(needs_layout_passes=False),
)
def _lookup(species_hbm, const_hbm, out_hbm,
            table_v, idx0, idx1, val0, val1,
            sin0, sin1, sout0, sout1):
    wid = lax.axis_index("s") * NC + lax.axis_index("c")
    base = wid * PER_W

    pltpu.sync_copy(const_hbm, table_v)

    idx = (idx0, idx1)
    val = (val0, val1)
    sin = (sin0, sin1)
    sout = (sout0, sout1)

    def in_copy(c, b):
        return pltpu.make_async_copy(
            species_hbm.at[pl.ds(base + c * CHUNK, CHUNK)], idx[b], sin[b])

    def out_copy(c, b):
        return pltpu.make_async_copy(
            val[b], out_hbm.at[pl.ds(base + c * CHUNK, CHUNK)], sout[b])

    def compute(b):
        idx_ref = idx[b]
        val_ref = val[b]

        @plsc.parallel_loop(0, CHUNK, L, unroll=8)
        def _(i):
            sl = pl.ds(i, L)
            val_ref[sl] = plsc.load_gather(table_v, [idx_ref[sl]])

    in_copy(0, 0).start()
    in_copy(1, 1).start()
    for c in range(NCHUNK):
        b = c & 1
        in_copy(c, b).wait()
        if c >= 2:
            out_copy(c - 2, b).wait()
        compute(b)
        out_copy(c, b).start()
        if c + 2 < NCHUNK:
            in_copy(c + 2, b).start()
    out_copy(NCHUNK - 2, 0).wait()
    out_copy(NCHUNK - 1, 1).wait()


def kernel(species, constant):
    return _lookup(species, constant)
